# baseline (speedup 1.0000x reference)
import functools

import jax
import jax.numpy as jnp
from jax import lax
from jax.experimental import pallas as pl
from jax.experimental.pallas import tpu as pltpu
from jax.experimental.pallas import tpu_sc as plsc

N = 10000
E = 320000
D = 128

NC = 2
NS = 16
NW = NC * NS
EPT = E // NW
CH = 40
NHALF = 2
HC = EPT // CH // NHALF
NBUF = 3
AHEAD = 2
RPT = 624

_mesh = plsc.VectorSubcoreMesh(core_axis_name="c", subcore_axis_name="s")


@functools.partial(
    pl.kernel,
    out_type=jax.ShapeDtypeStruct((NC, N, D), jnp.float32),
    mesh=_mesh,
    scratch_types=[
        pltpu.VMEM((HC, CH), jnp.int32),
        pltpu.VMEM((HC, CH), jnp.int32),
        pltpu.VMEM((NBUF, CH, D), jnp.float32),
        pltpu.VMEM_SHARED((N, D), jnp.float32),
        pltpu.SemaphoreType.DMA((NBUF,)),
        pltpu.SemaphoreType.DMA((NBUF,)),
        pltpu.SemaphoreType.DMA,
    ],
)
def _sc_aggregate(h_hbm, edges_hbm, out_hbm, src_v, dst_v, rows_v,
                  acc_sh, gsem, ssem, zsem):
    cid = lax.axis_index("c")
    sid = lax.axis_index("s")
    wid = sid * NC + cid

    cp_src = pltpu.async_copy(edges_hbm.at[0, wid, 0], src_v, zsem)
    cp_dst = pltpu.async_copy(edges_hbm.at[1, wid, 0], dst_v, zsem)

    def _zero_body(i, _):
        r = i // (D // 16)
        c = (i % (D // 16)) * 16
        rows_v[0, r, pl.ds(c, 16)] = jnp.zeros((16,), jnp.float32)
        return 0

    lax.fori_loop(0, CH * (D // 16), _zero_body, 0)

    base = sid * RPT
    for k in range(RPT // CH):
        pltpu.async_copy(rows_v.at[0], acc_sh.at[pl.ds(base + k * CH, CH)],
                         zsem)
    rem = RPT % CH
    if rem:
        pltpu.async_copy(rows_v.at[0, pl.ds(0, rem)],
                         acc_sh.at[pl.ds(base + (RPT // CH) * CH, rem)], zsem)

    @pl.when(sid == NS - 1)
    def _zero_tail():
        pltpu.async_copy(rows_v.at[0, pl.ds(0, N - NS * RPT)],
                         acc_sh.at[pl.ds(NS * RPT, N - NS * RPT)], zsem)

    cp_src.wait()
    cp_dst.wait()
    for k in range(RPT // CH):
        pltpu.make_async_copy(rows_v.at[0], acc_sh.at[pl.ds(base + k * CH, CH)],
                              zsem).wait()
    if rem:
        pltpu.make_async_copy(rows_v.at[0, pl.ds(0, rem)],
                              acc_sh.at[pl.ds(base + (RPT // CH) * CH, rem)],
                              zsem).wait()

    @pl.when(sid == NS - 1)
    def _drain_tail():
        pltpu.make_async_copy(rows_v.at[0, pl.ds(0, N - NS * RPT)],
                              acc_sh.at[pl.ds(NS * RPT, N - NS * RPT)],
                              zsem).wait()

    plsc.subcore_barrier()

    def _fire_gather(j, s):
        pltpu.async_copy(h_hbm.at[src_v.at[j]], rows_v.at[s], gsem.at[s])

    def _wait_gather(s):
        pltpu.make_async_copy(h_hbm.at[pl.ds(0, CH)], rows_v.at[s],
                              gsem.at[s]).wait()

    def _fire_scatter(j, s):
        pltpu.async_copy(rows_v.at[s], acc_sh.at[dst_v.at[j]], ssem.at[s],
                         add=True)

    def _wait_scatter(s):
        pltpu.make_async_copy(h_hbm.at[pl.ds(0, CH)], rows_v.at[s],
                              ssem.at[s]).wait()

    def _step(j, r, wait_sc, prefetch):
        _wait_gather(r)
        _fire_scatter(j, r)
        if prefetch:
            s_pre = (r + AHEAD) % NBUF
            if wait_sc:
                _wait_scatter(s_pre)
            _fire_gather(j + AHEAD, s_pre)

    _last_full = ((HC - 1 - NBUF) // NBUF) * NBUF + NBUF

    def _process_half(half):
        if half > 0:
            pltpu.sync_copy(edges_hbm.at[0, wid, half], src_v)
            pltpu.sync_copy(edges_hbm.at[1, wid, half], dst_v)
        for j in range(AHEAD):
            _fire_gather(j, j)
        for j in range(NBUF):
            _step(j, j, wait_sc=(j >= NBUF - AHEAD), prefetch=True)

        @pl.loop(NBUF, _last_full, step=NBUF)
        def _main(b):
            for r in range(NBUF):
                _step(b + r, r, wait_sc=True, prefetch=True)

        for j in range(_last_full, HC):
            _step(j, j % NBUF, wait_sc=True, prefetch=(j + AHEAD < HC))
        for s in range(NBUF):
            _wait_scatter(s)

    for half in range(NHALF):
        _process_half(half)

    plsc.subcore_barrier()

    pltpu.sync_copy(acc_sh.at[pl.ds(base, RPT)],
                    out_hbm.at[cid, pl.ds(base, RPT)])

    @pl.when(sid == NS - 1)
    def _export_tail():
        pltpu.sync_copy(acc_sh.at[pl.ds(NS * RPT, N - NS * RPT)],
                        out_hbm.at[cid, pl.ds(NS * RPT, N - NS * RPT)])


def _mlp_body(h_ref, p_ref, w1_ref, b1_ref, w2_ref, b2_ref, o_ref):
    s = h_ref[...] + p_ref[0] + p_ref[1]
    t = jnp.dot(s, w1_ref[...], preferred_element_type=jnp.float32) + b1_ref[...]
    t = jnp.maximum(t, 0.0)
    u = jnp.dot(t, w2_ref[...], preferred_element_type=jnp.float32) + b2_ref[...]
    o_ref[...] = jnp.maximum(u, 0.0)


def _mlp_final_body(h_ref, p_ref, w1_ref, b1_ref, w2_ref, b2_ref,
                    wc_ref, bc_ref, o_ref):
    s = h_ref[...] + p_ref[0] + p_ref[1]
    t = jnp.dot(s, w1_ref[...], preferred_element_type=jnp.float32) + b1_ref[...]
    t = jnp.maximum(t, 0.0)
    u = jnp.dot(t, w2_ref[...], preferred_element_type=jnp.float32) + b2_ref[...]
    u = jnp.maximum(u, 0.0)
    o_ref[...] = jnp.dot(u, wc_ref[...], preferred_element_type=jnp.float32) + bc_ref[...]


_BLK = 2000
_GRID = N // _BLK

_w_spec = pl.BlockSpec((D, D), lambda i: (0, 0))
_b_spec = pl.BlockSpec((1, D), lambda i: (0, 0))
_h_spec = pl.BlockSpec((_BLK, D), lambda i: (i, 0))
_p_spec = pl.BlockSpec((NC, _BLK, D), lambda i: (0, i, 0))

_mlp_call = pl.pallas_call(
    _mlp_body,
    out_shape=jax.ShapeDtypeStruct((N, D), jnp.float32),
    grid=(_GRID,),
    in_specs=[_h_spec, _p_spec, _w_spec, _b_spec, _w_spec, _b_spec],
    out_specs=_h_spec,
)

_mlp_final_call = pl.pallas_call(
    _mlp_final_body,
    out_shape=jax.ShapeDtypeStruct((N, D), jnp.float32),
    grid=(_GRID,),
    in_specs=[_h_spec, _p_spec, _w_spec, _b_spec, _w_spec, _b_spec,
              _w_spec, _b_spec],
    out_specs=_h_spec,
)


def kernel(x, edge_index, w1_0, b1_0, w2_0, b2_0, w1_1, b1_1, w2_1, b2_1,
           w1_2, b1_2, w2_2, b2_2, wc, bc):
    edges = edge_index.astype(jnp.int32).reshape(2, NW, NHALF, HC, CH)
    b1s = [b1_0.reshape(1, D), b1_1.reshape(1, D), b1_2.reshape(1, D)]
    b2s = [b2_0.reshape(1, D), b2_1.reshape(1, D), b2_2.reshape(1, D)]
    w1s = [w1_0, w1_1, w1_2]
    w2s = [w2_0, w2_1, w2_2]

    h = x
    for layer in range(3):
        parts = _sc_aggregate(h, edges)
        if layer < 2:
            h = _mlp_call(h, parts, w1s[layer], b1s[layer],
                          w2s[layer], b2s[layer])
        else:
            h = _mlp_final_call(h, parts, w1s[layer], b1s[layer],
                                w2s[layer], b2s[layer],
                                wc, bc.reshape(1, D))
    return h


# speedup vs baseline: 1.1036x; 1.1036x over previous
import functools

import jax
import jax.numpy as jnp
from jax import lax
from jax.experimental import pallas as pl
from jax.experimental.pallas import tpu as pltpu
from jax.experimental.pallas import tpu_sc as plsc

N = 10000
E = 320000
D = 128

NC = 2
NS = 16
NW = NC * NS
EPT = E // NW
CH = 40
NHALF = 2
HC = EPT // CH // NHALF
NBUF = 3
AHEAD = 2
RPT = 624

_mesh = plsc.VectorSubcoreMesh(core_axis_name="c", subcore_axis_name="s")


@functools.partial(
    pl.kernel,
    out_type=jax.ShapeDtypeStruct((NC, N, D), jnp.float32),
    mesh=_mesh,
    scratch_types=[
        pltpu.VMEM((HC, CH), jnp.int32),
        pltpu.VMEM((HC, CH), jnp.int32),
        pltpu.VMEM((NBUF, CH, D), jnp.float32),
        pltpu.VMEM_SHARED((N, D), jnp.float32),
        pltpu.SemaphoreType.DMA((NBUF,)),
        pltpu.SemaphoreType.DMA((NBUF,)),
        pltpu.SemaphoreType.DMA,
    ],
)
def _sc_aggregate(h_hbm, edges_hbm, out_hbm, src_v, dst_v, rows_v,
                  acc_sh, gsem, ssem, zsem):
    cid = lax.axis_index("c")
    sid = lax.axis_index("s")
    wid = sid * NC + cid

    cp_src = pltpu.async_copy(edges_hbm.at[0, wid, 0], src_v, zsem)
    cp_dst = pltpu.async_copy(edges_hbm.at[1, wid, 0], dst_v, zsem)

    def _zero_body(i, _):
        r = i // (D // 16)
        c = (i % (D // 16)) * 16
        rows_v[0, r, pl.ds(c, 16)] = jnp.zeros((16,), jnp.float32)
        return 0

    lax.fori_loop(0, CH * (D // 16), _zero_body, 0)

    base = sid * RPT
    for k in range(RPT // CH):
        pltpu.async_copy(rows_v.at[0], acc_sh.at[pl.ds(base + k * CH, CH)],
                         zsem)
    rem = RPT % CH
    if rem:
        pltpu.async_copy(rows_v.at[0, pl.ds(0, rem)],
                         acc_sh.at[pl.ds(base + (RPT // CH) * CH, rem)], zsem)

    @pl.when(sid == NS - 1)
    def _zero_tail():
        pltpu.async_copy(rows_v.at[0, pl.ds(0, N - NS * RPT)],
                         acc_sh.at[pl.ds(NS * RPT, N - NS * RPT)], zsem)

    cp_src.wait()
    cp_dst.wait()
    for k in range(RPT // CH):
        pltpu.make_async_copy(rows_v.at[0], acc_sh.at[pl.ds(base + k * CH, CH)],
                              zsem).wait()
    if rem:
        pltpu.make_async_copy(rows_v.at[0, pl.ds(0, rem)],
                              acc_sh.at[pl.ds(base + (RPT // CH) * CH, rem)],
                              zsem).wait()

    @pl.when(sid == NS - 1)
    def _drain_tail():
        pltpu.make_async_copy(rows_v.at[0, pl.ds(0, N - NS * RPT)],
                              acc_sh.at[pl.ds(NS * RPT, N - NS * RPT)],
                              zsem).wait()

    plsc.subcore_barrier()

    def _fire_gather(j, s):
        pltpu.async_copy(h_hbm.at[src_v.at[j]], rows_v.at[s], gsem.at[s])

    def _wait_gather(s):
        pltpu.make_async_copy(h_hbm.at[pl.ds(0, CH)], rows_v.at[s],
                              gsem.at[s]).wait()

    def _fire_scatter(j, s):
        pltpu.async_copy(rows_v.at[s], acc_sh.at[dst_v.at[j]], ssem.at[s],
                         add=True)

    def _wait_scatter(s):
        pltpu.make_async_copy(h_hbm.at[pl.ds(0, CH)], rows_v.at[s],
                              ssem.at[s]).wait()

    def _step(j, r, wait_sc, prefetch):
        if prefetch:
            s_pre = (r + AHEAD) % NBUF
            if wait_sc:
                _wait_scatter(s_pre)
            _fire_gather(j + AHEAD, s_pre)
        _wait_gather(r)
        _fire_scatter(j, r)

    _last_full = ((HC - 1 - NBUF) // NBUF) * NBUF + NBUF

    def _process_half(half):
        if half > 0:
            pltpu.sync_copy(edges_hbm.at[0, wid, half], src_v)
            pltpu.sync_copy(edges_hbm.at[1, wid, half], dst_v)
        for j in range(AHEAD):
            _fire_gather(j, j)
        for j in range(NBUF):
            _step(j, j, wait_sc=(j >= NBUF - AHEAD), prefetch=True)

        @pl.loop(NBUF, _last_full, step=NBUF)
        def _main(b):
            for r in range(NBUF):
                _step(b + r, r, wait_sc=True, prefetch=True)

        for j in range(_last_full, HC):
            _step(j, j % NBUF, wait_sc=True, prefetch=(j + AHEAD < HC))
        for s in range(NBUF):
            _wait_scatter(s)

    for half in range(NHALF):
        _process_half(half)

    plsc.subcore_barrier()

    pltpu.sync_copy(acc_sh.at[pl.ds(base, RPT)],
                    out_hbm.at[cid, pl.ds(base, RPT)])

    @pl.when(sid == NS - 1)
    def _export_tail():
        pltpu.sync_copy(acc_sh.at[pl.ds(NS * RPT, N - NS * RPT)],
                        out_hbm.at[cid, pl.ds(NS * RPT, N - NS * RPT)])


def _mlp_body(h_ref, p_ref, w1_ref, b1_ref, w2_ref, b2_ref, o_ref):
    s = h_ref[...] + p_ref[0] + p_ref[1]
    t = jnp.dot(s, w1_ref[...], preferred_element_type=jnp.float32) + b1_ref[...]
    t = jnp.maximum(t, 0.0)
    u = jnp.dot(t, w2_ref[...], preferred_element_type=jnp.float32) + b2_ref[...]
    o_ref[...] = jnp.maximum(u, 0.0)


def _mlp_final_body(h_ref, p_ref, w1_ref, b1_ref, w2_ref, b2_ref,
                    wc_ref, bc_ref, o_ref):
    s = h_ref[...] + p_ref[0] + p_ref[1]
    t = jnp.dot(s, w1_ref[...], preferred_element_type=jnp.float32) + b1_ref[...]
    t = jnp.maximum(t, 0.0)
    u = jnp.dot(t, w2_ref[...], preferred_element_type=jnp.float32) + b2_ref[...]
    u = jnp.maximum(u, 0.0)
    o_ref[...] = jnp.dot(u, wc_ref[...], preferred_element_type=jnp.float32) + bc_ref[...]


_BLK = 2000
_GRID = N // _BLK

_w_spec = pl.BlockSpec((D, D), lambda i: (0, 0))
_b_spec = pl.BlockSpec((1, D), lambda i: (0, 0))
_h_spec = pl.BlockSpec((_BLK, D), lambda i: (i, 0))
_p_spec = pl.BlockSpec((NC, _BLK, D), lambda i: (0, i, 0))

_mlp_call = pl.pallas_call(
    _mlp_body,
    out_shape=jax.ShapeDtypeStruct((N, D), jnp.float32),
    grid=(_GRID,),
    in_specs=[_h_spec, _p_spec, _w_spec, _b_spec, _w_spec, _b_spec],
    out_specs=_h_spec,
)

_mlp_final_call = pl.pallas_call(
    _mlp_final_body,
    out_shape=jax.ShapeDtypeStruct((N, D), jnp.float32),
    grid=(_GRID,),
    in_specs=[_h_spec, _p_spec, _w_spec, _b_spec, _w_spec, _b_spec,
              _w_spec, _b_spec],
    out_specs=_h_spec,
)


def kernel(x, edge_index, w1_0, b1_0, w2_0, b2_0, w1_1, b1_1, w2_1, b2_1,
           w1_2, b1_2, w2_2, b2_2, wc, bc):
    edges = edge_index.astype(jnp.int32).reshape(2, NW, NHALF, HC, CH)
    b1s = [b1_0.reshape(1, D), b1_1.reshape(1, D), b1_2.reshape(1, D)]
    b2s = [b2_0.reshape(1, D), b2_1.reshape(1, D), b2_2.reshape(1, D)]
    w1s = [w1_0, w1_1, w1_2]
    w2s = [w2_0, w2_1, w2_2]

    h = x
    for layer in range(3):
        parts = _sc_aggregate(h, edges)
        if layer < 2:
            h = _mlp_call(h, parts, w1s[layer], b1s[layer],
                          w2s[layer], b2s[layer])
        else:
            h = _mlp_final_call(h, parts, w1s[layer], b1s[layer],
                                w2s[layer], b2s[layer],
                                wc, bc.reshape(1, D))
    return h


# speedup vs baseline: 1.1147x; 1.0101x over previous
import functools

import jax
import jax.numpy as jnp
from jax import lax
from jax.experimental import pallas as pl
from jax.experimental.pallas import tpu as pltpu
from jax.experimental.pallas import tpu_sc as plsc

N = 10000
E = 320000
D = 128

NC = 2
NS = 16
NW = NC * NS
EPT = E // NW
CH = 40
NHALF = 2
HC = EPT // CH // NHALF
NBUF = 3
AHEAD = 2
RPT = 624

_mesh = plsc.VectorSubcoreMesh(core_axis_name="c", subcore_axis_name="s")


@functools.partial(
    pl.kernel,
    out_type=jax.ShapeDtypeStruct((NC, N, D), jnp.float32),
    mesh=_mesh,
    scratch_types=[
        pltpu.VMEM((HC, CH), jnp.int32),
        pltpu.VMEM((HC, CH), jnp.int32),
        pltpu.VMEM((NBUF, CH, D), jnp.float32),
        pltpu.VMEM_SHARED((N, D), jnp.float32),
        pltpu.SemaphoreType.DMA((NBUF,)),
        pltpu.SemaphoreType.DMA((NBUF,)),
        pltpu.SemaphoreType.DMA,
    ],
)
def _sc_aggregate(h_hbm, edges_hbm, out_hbm, src_v, dst_v, rows_v,
                  acc_sh, gsem, ssem, zsem):
    cid = lax.axis_index("c")
    sid = lax.axis_index("s")
    wid = sid * NC + cid

    cp_src = pltpu.async_copy(edges_hbm.at[0, wid, 0], src_v, zsem)
    cp_dst = pltpu.async_copy(edges_hbm.at[1, wid, 0], dst_v, zsem)

    def _zero_body(i, _):
        r = i // (D // 16)
        c = (i % (D // 16)) * 16
        rows_v[0, r, pl.ds(c, 16)] = jnp.zeros((16,), jnp.float32)
        return 0

    lax.fori_loop(0, CH * (D // 16), _zero_body, 0)

    base = sid * RPT
    for k in range(RPT // CH):
        pltpu.async_copy(rows_v.at[0], acc_sh.at[pl.ds(base + k * CH, CH)],
                         zsem)
    rem = RPT % CH
    if rem:
        pltpu.async_copy(rows_v.at[0, pl.ds(0, rem)],
                         acc_sh.at[pl.ds(base + (RPT // CH) * CH, rem)], zsem)

    @pl.when(sid == NS - 1)
    def _zero_tail():
        pltpu.async_copy(rows_v.at[0, pl.ds(0, N - NS * RPT)],
                         acc_sh.at[pl.ds(NS * RPT, N - NS * RPT)], zsem)

    cp_src.wait()
    cp_dst.wait()
    for k in range(RPT // CH):
        pltpu.make_async_copy(rows_v.at[0], acc_sh.at[pl.ds(base + k * CH, CH)],
                              zsem).wait()
    if rem:
        pltpu.make_async_copy(rows_v.at[0, pl.ds(0, rem)],
                              acc_sh.at[pl.ds(base + (RPT // CH) * CH, rem)],
                              zsem).wait()

    @pl.when(sid == NS - 1)
    def _drain_tail():
        pltpu.make_async_copy(rows_v.at[0, pl.ds(0, N - NS * RPT)],
                              acc_sh.at[pl.ds(NS * RPT, N - NS * RPT)],
                              zsem).wait()

    plsc.subcore_barrier()

    def _fire_gather(j, s):
        pltpu.async_copy(h_hbm.at[src_v.at[j]], rows_v.at[s], gsem.at[s])

    def _wait_gather(s):
        pltpu.make_async_copy(h_hbm.at[pl.ds(0, CH)], rows_v.at[s],
                              gsem.at[s]).wait()

    def _fire_scatter(j, s):
        pltpu.async_copy(rows_v.at[s], acc_sh.at[dst_v.at[j]], ssem.at[s],
                         add=True)

    def _wait_scatter(s):
        pltpu.make_async_copy(h_hbm.at[pl.ds(0, CH)], rows_v.at[s],
                              ssem.at[s]).wait()

    def _step(j, r, wait_sc, prefetch):
        if prefetch:
            s_pre = (r + AHEAD) % NBUF
            if wait_sc:
                _wait_scatter(s_pre)
            _fire_gather(j + AHEAD, s_pre)
        _wait_gather(r)
        _fire_scatter(j, r)

    _last_full = ((HC - 1 - NBUF) // NBUF) * NBUF + NBUF

    def _process_half(half):
        if half > 0:
            pltpu.sync_copy(edges_hbm.at[0, wid, half], src_v)
            pltpu.sync_copy(edges_hbm.at[1, wid, half], dst_v)
        for j in range(AHEAD):
            _fire_gather(j, j)
        for j in range(NBUF):
            _step(j, j, wait_sc=(j >= NBUF - AHEAD), prefetch=True)

        @pl.loop(NBUF, _last_full, step=NBUF)
        def _main(b):
            for r in range(NBUF):
                _step(b + r, r, wait_sc=True, prefetch=True)

        for j in range(_last_full, HC):
            _step(j, j % NBUF, wait_sc=True, prefetch=(j + AHEAD < HC))
        for s in range(NBUF):
            _wait_scatter(s)

    for half in range(NHALF):
        _process_half(half)

    plsc.subcore_barrier()

    pltpu.sync_copy(acc_sh.at[pl.ds(base, RPT)],
                    out_hbm.at[cid, pl.ds(base, RPT)])

    @pl.when(sid == NS - 1)
    def _export_tail():
        pltpu.sync_copy(acc_sh.at[pl.ds(NS * RPT, N - NS * RPT)],
                        out_hbm.at[cid, pl.ds(NS * RPT, N - NS * RPT)])


def _mlp_body(h_ref, p_ref, w1_ref, b1_ref, w2_ref, b2_ref, o_ref):
    s = h_ref[...] + p_ref[0] + p_ref[1]
    t = jnp.dot(s, w1_ref[...], preferred_element_type=jnp.float32) + b1_ref[...]
    t = jnp.maximum(t, 0.0)
    u = jnp.dot(t, w2_ref[...], preferred_element_type=jnp.float32) + b2_ref[...]
    o_ref[...] = jnp.maximum(u, 0.0)


def _mlp_final_body(h_ref, p_ref, w1_ref, b1_ref, w2_ref, b2_ref,
                    wc_ref, bc_ref, o_ref):
    s = h_ref[...] + p_ref[0] + p_ref[1]
    t = jnp.dot(s, w1_ref[...], preferred_element_type=jnp.float32) + b1_ref[...]
    t = jnp.maximum(t, 0.0)
    u = jnp.dot(t, w2_ref[...], preferred_element_type=jnp.float32) + b2_ref[...]
    u = jnp.maximum(u, 0.0)
    o_ref[...] = jnp.dot(u, wc_ref[...], preferred_element_type=jnp.float32) + bc_ref[...]


_BLK = 5000
_GRID = N // _BLK

_w_spec = pl.BlockSpec((D, D), lambda i: (0, 0))
_b_spec = pl.BlockSpec((1, D), lambda i: (0, 0))
_h_spec = pl.BlockSpec((_BLK, D), lambda i: (i, 0))
_p_spec = pl.BlockSpec((NC, _BLK, D), lambda i: (0, i, 0))

_mlp_call = pl.pallas_call(
    _mlp_body,
    out_shape=jax.ShapeDtypeStruct((N, D), jnp.float32),
    grid=(_GRID,),
    in_specs=[_h_spec, _p_spec, _w_spec, _b_spec, _w_spec, _b_spec],
    out_specs=_h_spec,
)

_mlp_final_call = pl.pallas_call(
    _mlp_final_body,
    out_shape=jax.ShapeDtypeStruct((N, D), jnp.float32),
    grid=(_GRID,),
    in_specs=[_h_spec, _p_spec, _w_spec, _b_spec, _w_spec, _b_spec,
              _w_spec, _b_spec],
    out_specs=_h_spec,
)


def kernel(x, edge_index, w1_0, b1_0, w2_0, b2_0, w1_1, b1_1, w2_1, b2_1,
           w1_2, b1_2, w2_2, b2_2, wc, bc):
    edges = edge_index.astype(jnp.int32).reshape(2, NW, NHALF, HC, CH)
    b1s = [b1_0.reshape(1, D), b1_1.reshape(1, D), b1_2.reshape(1, D)]
    b2s = [b2_0.reshape(1, D), b2_1.reshape(1, D), b2_2.reshape(1, D)]
    w1s = [w1_0, w1_1, w1_2]
    w2s = [w2_0, w2_1, w2_2]

    h = x
    for layer in range(3):
        parts = _sc_aggregate(h, edges)
        if layer < 2:
            h = _mlp_call(h, parts, w1s[layer], b1s[layer],
                          w2s[layer], b2s[layer])
        else:
            h = _mlp_final_call(h, parts, w1s[layer], b1s[layer],
                                w2s[layer], b2s[layer],
                                wc, bc.reshape(1, D))
    return h


# speedup vs baseline: 1.1208x; 1.0054x over previous
import functools

import jax
import jax.numpy as jnp
from jax import lax
from jax.experimental import pallas as pl
from jax.experimental.pallas import tpu as pltpu
from jax.experimental.pallas import tpu_sc as plsc

N = 10000
E = 320000
D = 128

NC = 2
NS = 16
NW = NC * NS
EPT = E // NW
CH = 40
NHALF = 2
HC = EPT // CH // NHALF
NBUF = 3
AHEAD = 2
RPT = 624

_mesh = plsc.VectorSubcoreMesh(core_axis_name="c", subcore_axis_name="s")


@functools.partial(
    pl.kernel,
    out_type=jax.ShapeDtypeStruct((NC, N, D), jnp.float32),
    mesh=_mesh,
    scratch_types=[
        pltpu.VMEM((HC, CH), jnp.int32),
        pltpu.VMEM((HC, CH), jnp.int32),
        pltpu.VMEM((NBUF, CH, D), jnp.float32),
        pltpu.VMEM_SHARED((N, D), jnp.float32),
        pltpu.SemaphoreType.DMA((NBUF,)),
        pltpu.SemaphoreType.DMA((NBUF,)),
        pltpu.SemaphoreType.DMA,
    ],
)
def _sc_aggregate(h_hbm, edges_hbm, out_hbm, src_v, dst_v, rows_v,
                  acc_sh, gsem, ssem, zsem):
    cid = lax.axis_index("c")
    sid = lax.axis_index("s")
    wid = sid * NC + cid

    cp_src = pltpu.async_copy(edges_hbm.at[0, wid, 0], src_v, zsem)
    cp_dst = pltpu.async_copy(edges_hbm.at[1, wid, 0], dst_v, zsem)

    def _zero_body(i, _):
        r = i // (D // 16)
        c = (i % (D // 16)) * 16
        rows_v[0, r, pl.ds(c, 16)] = jnp.zeros((16,), jnp.float32)
        return 0

    lax.fori_loop(0, CH * (D // 16), _zero_body, 0)

    base = sid * RPT
    for k in range(RPT // CH):
        pltpu.async_copy(rows_v.at[0], acc_sh.at[pl.ds(base + k * CH, CH)],
                         zsem)
    rem = RPT % CH
    if rem:
        pltpu.async_copy(rows_v.at[0, pl.ds(0, rem)],
                         acc_sh.at[pl.ds(base + (RPT // CH) * CH, rem)], zsem)

    @pl.when(sid == NS - 1)
    def _zero_tail():
        pltpu.async_copy(rows_v.at[0, pl.ds(0, N - NS * RPT)],
                         acc_sh.at[pl.ds(NS * RPT, N - NS * RPT)], zsem)

    cp_src.wait()
    cp_dst.wait()
    for k in range(RPT // CH):
        pltpu.make_async_copy(rows_v.at[0], acc_sh.at[pl.ds(base + k * CH, CH)],
                              zsem).wait()
    if rem:
        pltpu.make_async_copy(rows_v.at[0, pl.ds(0, rem)],
                              acc_sh.at[pl.ds(base + (RPT // CH) * CH, rem)],
                              zsem).wait()

    @pl.when(sid == NS - 1)
    def _drain_tail():
        pltpu.make_async_copy(rows_v.at[0, pl.ds(0, N - NS * RPT)],
                              acc_sh.at[pl.ds(NS * RPT, N - NS * RPT)],
                              zsem).wait()

    def _fire_gather(j, s):
        pltpu.async_copy(h_hbm.at[src_v.at[j]], rows_v.at[s], gsem.at[s])

    def _wait_gather(s):
        pltpu.make_async_copy(h_hbm.at[pl.ds(0, CH)], rows_v.at[s],
                              gsem.at[s]).wait()

    def _fire_scatter(j, s):
        pltpu.async_copy(rows_v.at[s], acc_sh.at[dst_v.at[j]], ssem.at[s],
                         add=True)

    def _wait_scatter(s):
        pltpu.make_async_copy(h_hbm.at[pl.ds(0, CH)], rows_v.at[s],
                              ssem.at[s]).wait()

    def _step(j, r, wait_sc, prefetch):
        if prefetch:
            s_pre = (r + AHEAD) % NBUF
            if wait_sc:
                _wait_scatter(s_pre)
            _fire_gather(j + AHEAD, s_pre)
        _wait_gather(r)
        _fire_scatter(j, r)

    _last_full = ((HC - 1 - NBUF) // NBUF) * NBUF + NBUF

    def _process_half(half, prologue_fired):
        if not prologue_fired:
            for j in range(AHEAD):
                _fire_gather(j, j)
        for j in range(NBUF):
            _step(j, j, wait_sc=(j >= NBUF - AHEAD), prefetch=True)

        @pl.loop(NBUF, _last_full, step=NBUF)
        def _main(b):
            for r in range(NBUF):
                _step(b + r, r, wait_sc=True, prefetch=True)

        for j in range(_last_full, HC):
            _step(j, j % NBUF, wait_sc=True, prefetch=(j + AHEAD < HC))
        if half + 1 < NHALF:
            cp = pltpu.async_copy(edges_hbm.at[0, wid, half + 1], src_v, zsem)
        for s in range(NBUF):
            _wait_scatter(s)
        if half + 1 < NHALF:
            pltpu.sync_copy(edges_hbm.at[1, wid, half + 1], dst_v)
            cp.wait()

    for j in range(AHEAD):
        _fire_gather(j, j)
    plsc.subcore_barrier()

    for half in range(NHALF):
        _process_half(half, prologue_fired=(half == 0))

    plsc.subcore_barrier()

    pltpu.sync_copy(acc_sh.at[pl.ds(base, RPT)],
                    out_hbm.at[cid, pl.ds(base, RPT)])

    @pl.when(sid == NS - 1)
    def _export_tail():
        pltpu.sync_copy(acc_sh.at[pl.ds(NS * RPT, N - NS * RPT)],
                        out_hbm.at[cid, pl.ds(NS * RPT, N - NS * RPT)])


def _mlp_body(h_ref, p_ref, w1_ref, b1_ref, w2_ref, b2_ref, o_ref):
    s = h_ref[...] + p_ref[0] + p_ref[1]
    t = jnp.dot(s, w1_ref[...], preferred_element_type=jnp.float32) + b1_ref[...]
    t = jnp.maximum(t, 0.0)
    u = jnp.dot(t, w2_ref[...], preferred_element_type=jnp.float32) + b2_ref[...]
    o_ref[...] = jnp.maximum(u, 0.0)


def _mlp_final_body(h_ref, p_ref, w1_ref, b1_ref, w2_ref, b2_ref,
                    wc_ref, bc_ref, o_ref):
    s = h_ref[...] + p_ref[0] + p_ref[1]
    t = jnp.dot(s, w1_ref[...], preferred_element_type=jnp.float32) + b1_ref[...]
    t = jnp.maximum(t, 0.0)
    u = jnp.dot(t, w2_ref[...], preferred_element_type=jnp.float32) + b2_ref[...]
    u = jnp.maximum(u, 0.0)
    o_ref[...] = jnp.dot(u, wc_ref[...], preferred_element_type=jnp.float32) + bc_ref[...]


_BLK = 5000
_GRID = N // _BLK

_w_spec = pl.BlockSpec((D, D), lambda i: (0, 0))
_b_spec = pl.BlockSpec((1, D), lambda i: (0, 0))
_h_spec = pl.BlockSpec((_BLK, D), lambda i: (i, 0))
_p_spec = pl.BlockSpec((NC, _BLK, D), lambda i: (0, i, 0))

_mlp_call = pl.pallas_call(
    _mlp_body,
    out_shape=jax.ShapeDtypeStruct((N, D), jnp.float32),
    grid=(_GRID,),
    in_specs=[_h_spec, _p_spec, _w_spec, _b_spec, _w_spec, _b_spec],
    out_specs=_h_spec,
)

_mlp_final_call = pl.pallas_call(
    _mlp_final_body,
    out_shape=jax.ShapeDtypeStruct((N, D), jnp.float32),
    grid=(_GRID,),
    in_specs=[_h_spec, _p_spec, _w_spec, _b_spec, _w_spec, _b_spec,
              _w_spec, _b_spec],
    out_specs=_h_spec,
)


def kernel(x, edge_index, w1_0, b1_0, w2_0, b2_0, w1_1, b1_1, w2_1, b2_1,
           w1_2, b1_2, w2_2, b2_2, wc, bc):
    edges = edge_index.astype(jnp.int32).reshape(2, NW, NHALF, HC, CH)
    b1s = [b1_0.reshape(1, D), b1_1.reshape(1, D), b1_2.reshape(1, D)]
    b2s = [b2_0.reshape(1, D), b2_1.reshape(1, D), b2_2.reshape(1, D)]
    w1s = [w1_0, w1_1, w1_2]
    w2s = [w2_0, w2_1, w2_2]

    h = x
    for layer in range(3):
        parts = _sc_aggregate(h, edges)
        if layer < 2:
            h = _mlp_call(h, parts, w1s[layer], b1s[layer],
                          w2s[layer], b2s[layer])
        else:
            h = _mlp_final_call(h, parts, w1s[layer], b1s[layer],
                                w2s[layer], b2s[layer],
                                wc, bc.reshape(1, D))
    return h
